# baseline (speedup 1.0000x reference)
import jax
import jax.numpy as jnp
from jax.experimental import pallas as pl
from jax.experimental.pallas import tpu as pltpu

Q = 4
K = 1024
D = 256
ROWS = 8 * 1024
TILE = 1024
NCH = 4
HALF = TILE // NCH


_TDN = (((1,), (1,)), ((), ()))


def _rvq_body(x_ref, cb_ref, q_ref, idx_ref, loss_ref):
    i = pl.program_id(0)
    cb = cb_ref[...]
    cbsq = cb * cb
    ones_row = jnp.ones((1, D), jnp.float32)
    cc = jax.lax.dot_general(ones_row, cbsq, _TDN,
                             preferred_element_type=jnp.float32,
                             precision=jax.lax.Precision.HIGHEST)
    cb2 = -2.0 * cb
    lane = jax.lax.broadcasted_iota(jnp.int32, (HALF, K), 1)
    scale = 1.0 / (ROWS * D)
    cb_hi = cb.astype(jnp.bfloat16)
    res1 = cb - cb_hi.astype(jnp.float32)
    cb_mid = res1.astype(jnp.bfloat16)
    cb_lo = (res1 - cb_mid.astype(jnp.float32)).astype(jnp.bfloat16)

    partials = [0.0] * Q
    sls = [slice(h * HALF, (h + 1) * HALF) for h in range(NCH)]
    r = [x_ref[s, :] for s in sls]
    for k in range(Q):
        xc2 = [jax.lax.dot_general(r[h], cb2, _TDN,
                                   preferred_element_type=jnp.float32)
               for h in range(NCH)]
        d = [xc2[h] + cc for h in range(NCH)]
        m = [jnp.min(d[h], axis=1, keepdims=True) for h in range(NCH)]
        idx = [jnp.min(jnp.where(d[h] == m[h], lane, K), axis=1, keepdims=True)
               for h in range(NCH)]
        onehot = [(lane == idx[h]).astype(jnp.bfloat16) for h in range(NCH)]
        q = [(jnp.dot(onehot[h], cb_hi, preferred_element_type=jnp.float32)
              + jnp.dot(onehot[h], cb_mid, preferred_element_type=jnp.float32)
              + jnp.dot(onehot[h], cb_lo, preferred_element_type=jnp.float32))
             for h in range(NCH)]
        t = [q[h] - r[h] for h in range(NCH)]
        for h in range(NCH):
            partials[k] = partials[k] + jnp.sum(t[h] * t[h]) * scale
        q_st = [r[h] + t[h] for h in range(NCH)]
        for h in range(NCH):
            q_ref[k, sls[h], :] = q_st[h]
            idx_ref[k:k + 1, sls[h]] = jnp.swapaxes(idx[h], 0, 1)
        r = [r[h] - q_st[h] for h in range(NCH)]

    @pl.when(i == 0)
    def _init():
        for k in range(Q):
            loss_ref[k] = partials[k]

    @pl.when(i > 0)
    def _acc():
        for k in range(Q):
            loss_ref[k] = loss_ref[k] + partials[k]


def kernel(x, codebook):
    b, n, d = x.shape
    flat = x.reshape(ROWS, D)
    grid = (ROWS // TILE,)
    quantized, idx_t, losses = pl.pallas_call(
        _rvq_body,
        grid=grid,
        in_specs=[
            pl.BlockSpec((TILE, D), lambda i: (i, 0)),
            pl.BlockSpec((K, D), lambda i: (0, 0)),
        ],
        out_specs=[
            pl.BlockSpec((Q, TILE, D), lambda i: (0, i, 0)),
            pl.BlockSpec((Q, TILE), lambda i: (0, i)),
            pl.BlockSpec(memory_space=pltpu.SMEM),
        ],
        out_shape=[
            jax.ShapeDtypeStruct((Q, ROWS, D), jnp.float32),
            jax.ShapeDtypeStruct((Q, ROWS), jnp.int32),
            jax.ShapeDtypeStruct((Q,), jnp.float32),
        ],
    )(flat, codebook)
    quantized = quantized.reshape(Q, b, n, d)
    indices = idx_t.reshape(Q, b, n)
    return quantized, indices, losses


# speedup vs baseline: 1.0594x; 1.0594x over previous
import jax
import jax.numpy as jnp
from jax.experimental import pallas as pl
from jax.experimental.pallas import tpu as pltpu

Q = 4
K = 1024
D = 256
ROWS = 8 * 1024
TILE = 1024
NCH = 4
HALF = TILE // NCH


def _rvq_body(x_ref, cb_ref, cbt_ref, q_ref, idx_ref, loss_ref):
    i = pl.program_id(0)
    cb = cb_ref[...]
    cbt = cbt_ref[...]
    cc = jnp.sum(cbt * cbt, axis=0, keepdims=True)
    cbt2 = -2.0 * cbt
    lane = jax.lax.broadcasted_iota(jnp.int32, (HALF, K), 1).astype(jnp.float32)
    scale = 1.0 / (ROWS * D)
    cb_hi = cb.astype(jnp.bfloat16)
    res1 = cb - cb_hi.astype(jnp.float32)
    cb_mid = res1.astype(jnp.bfloat16)
    cb_lo = (res1 - cb_mid.astype(jnp.float32)).astype(jnp.bfloat16)

    partials = [0.0] * Q
    sls = [slice(h * HALF, (h + 1) * HALF) for h in range(NCH)]
    r = [x_ref[s, :] for s in sls]
    for k in range(Q):
        xc2 = [jnp.dot(r[h], cbt2, preferred_element_type=jnp.float32)
               for h in range(NCH)]
        d = [xc2[h] + cc for h in range(NCH)]
        m = [jnp.min(d[h], axis=1, keepdims=True) for h in range(NCH)]
        idx = [jnp.min(jnp.where(d[h] == m[h], lane, float(K)),
                       axis=1, keepdims=True)
               for h in range(NCH)]
        onehot = [(lane == idx[h]).astype(jnp.bfloat16) for h in range(NCH)]
        q = [(jnp.dot(onehot[h], cb_hi, preferred_element_type=jnp.float32)
              + jnp.dot(onehot[h], cb_mid, preferred_element_type=jnp.float32)
              + jnp.dot(onehot[h], cb_lo, preferred_element_type=jnp.float32))
             for h in range(NCH)]
        t = [q[h] - r[h] for h in range(NCH)]
        for h in range(NCH):
            partials[k] = partials[k] + jnp.sum(t[h] * t[h]) * scale
        q_st = [r[h] + t[h] for h in range(NCH)]
        for h in range(NCH):
            q_ref[k, sls[h], :] = q_st[h]
            idx_ref[sls[h], k:k + 1] = idx[h].astype(jnp.int32)
        r = [r[h] - q_st[h] for h in range(NCH)]

    @pl.when(i == 0)
    def _init():
        for k in range(Q):
            loss_ref[k] = partials[k]

    @pl.when(i > 0)
    def _acc():
        for k in range(Q):
            loss_ref[k] = loss_ref[k] + partials[k]


def kernel(x, codebook):
    b, n, d = x.shape
    flat = x.reshape(ROWS, D)
    grid = (ROWS // TILE,)
    quantized, idx_t, losses = pl.pallas_call(
        _rvq_body,
        grid=grid,
        in_specs=[
            pl.BlockSpec((TILE, D), lambda i: (i, 0)),
            pl.BlockSpec((K, D), lambda i: (0, 0)),
            pl.BlockSpec((D, K), lambda i: (0, 0)),
        ],
        out_specs=[
            pl.BlockSpec((Q, TILE, D), lambda i: (0, i, 0)),
            pl.BlockSpec((TILE, Q), lambda i: (i, 0)),
            pl.BlockSpec(memory_space=pltpu.SMEM),
        ],
        out_shape=[
            jax.ShapeDtypeStruct((Q, ROWS, D), jnp.float32),
            jax.ShapeDtypeStruct((ROWS, Q), jnp.int32),
            jax.ShapeDtypeStruct((Q,), jnp.float32),
        ],
    )(flat, codebook, codebook.T)
    quantized = quantized.reshape(Q, b, n, d)
    indices = idx_t.T.reshape(Q, b, n)
    return quantized, indices, losses


# speedup vs baseline: 1.1003x; 1.0385x over previous
import jax
import jax.numpy as jnp
from jax.experimental import pallas as pl
from jax.experimental.pallas import tpu as pltpu

Q = 4
K = 1024
D = 256
ROWS = 8 * 1024
TILE = 2048
NCH = 4
HALF = TILE // NCH


def _rvq_body(x_ref, cb_ref, cbt_ref, q_ref, idx_ref, loss_ref):
    i = pl.program_id(0)
    cb = cb_ref[...]
    cbt = cbt_ref[...]
    cc = jnp.sum(cbt * cbt, axis=0, keepdims=True)
    cbt2 = -2.0 * cbt
    lane = jax.lax.broadcasted_iota(jnp.int32, (HALF, K), 1).astype(jnp.float32)
    scale = 1.0 / (ROWS * D)
    cb_hi = cb.astype(jnp.bfloat16)
    res1 = cb - cb_hi.astype(jnp.float32)
    cb_mid = res1.astype(jnp.bfloat16)
    cb_lo = (res1 - cb_mid.astype(jnp.float32)).astype(jnp.bfloat16)

    partials = [0.0] * Q
    sls = [slice(h * HALF, (h + 1) * HALF) for h in range(NCH)]
    r = [x_ref[s, :] for s in sls]
    for k in range(Q):
        xc2 = [jnp.dot(r[h], cbt2, preferred_element_type=jnp.float32)
               for h in range(NCH)]
        d = [xc2[h] + cc for h in range(NCH)]
        m = [jnp.min(d[h], axis=1, keepdims=True) for h in range(NCH)]
        idx = [jnp.min(jnp.where(d[h] == m[h], lane, float(K)),
                       axis=1, keepdims=True)
               for h in range(NCH)]
        onehot = [(lane == idx[h]).astype(jnp.bfloat16) for h in range(NCH)]
        q = [(jnp.dot(onehot[h], cb_hi, preferred_element_type=jnp.float32)
              + jnp.dot(onehot[h], cb_mid, preferred_element_type=jnp.float32)
              + jnp.dot(onehot[h], cb_lo, preferred_element_type=jnp.float32))
             for h in range(NCH)]
        t = [q[h] - r[h] for h in range(NCH)]
        for h in range(NCH):
            partials[k] = partials[k] + jnp.sum(t[h] * t[h]) * scale
        q_st = [r[h] + t[h] for h in range(NCH)]
        for h in range(NCH):
            q_ref[k, sls[h], :] = q_st[h]
            idx_ref[sls[h], k:k + 1] = idx[h].astype(jnp.int32)
        r = [r[h] - q_st[h] for h in range(NCH)]

    @pl.when(i == 0)
    def _init():
        for k in range(Q):
            loss_ref[k] = partials[k]

    @pl.when(i > 0)
    def _acc():
        for k in range(Q):
            loss_ref[k] = loss_ref[k] + partials[k]


def kernel(x, codebook):
    b, n, d = x.shape
    flat = x.reshape(ROWS, D)
    grid = (ROWS // TILE,)
    quantized, idx_t, losses = pl.pallas_call(
        _rvq_body,
        grid=grid,
        in_specs=[
            pl.BlockSpec((TILE, D), lambda i: (i, 0)),
            pl.BlockSpec((K, D), lambda i: (0, 0)),
            pl.BlockSpec((D, K), lambda i: (0, 0)),
        ],
        out_specs=[
            pl.BlockSpec((Q, TILE, D), lambda i: (0, i, 0)),
            pl.BlockSpec((TILE, Q), lambda i: (i, 0)),
            pl.BlockSpec(memory_space=pltpu.SMEM),
        ],
        out_shape=[
            jax.ShapeDtypeStruct((Q, ROWS, D), jnp.float32),
            jax.ShapeDtypeStruct((ROWS, Q), jnp.int32),
            jax.ShapeDtypeStruct((Q,), jnp.float32),
        ],
    )(flat, codebook, codebook.T)
    quantized = quantized.reshape(Q, b, n, d)
    indices = idx_t.T.reshape(Q, b, n)
    return quantized, indices, losses
